# baseline (speedup 1.0000x reference)
import functools

import jax
import jax.numpy as jnp
from jax import lax
from jax.experimental import pallas as pl
from jax.experimental.pallas import tpu as pltpu
from jax.experimental.pallas import tpu_sc as plsc

EMBED_DIM = 64
SEQ = 50
NUM_WORKERS = 32
GROUP = 16


def _make_gather(n_sent: int, n_chunks: int):
    mesh = plsc.VectorSubcoreMesh(core_axis_name="c", subcore_axis_name="s")
    s_per_w = n_sent // NUM_WORKERS
    rows_per_chunk = GROUP * SEQ

    @functools.partial(
        pl.kernel,
        mesh=mesh,
        compiler_params=pltpu.CompilerParams(use_tc_tiling_on_sc=False),
        out_type=jax.ShapeDtypeStruct((n_sent, SEQ, EMBED_DIM), jnp.float32),
        scratch_types=[
            pltpu.VMEM((rows_per_chunk,), jnp.int32),
            pltpu.VMEM((rows_per_chunk, EMBED_DIM), jnp.float32),
            pltpu.SemaphoreType.DMA,
        ],
    )
    def gather(table_hbm, idx_hbm, out_hbm, idx_v, rows_v, sem):
        wid = lax.axis_index("s") * 2 + lax.axis_index("c")
        base_s = wid * s_per_w

        def body(g, carry):
            s0 = base_s + g * GROUP
            row_off = s0 * SEQ
            pltpu.sync_copy(idx_hbm.at[pl.ds(row_off, rows_per_chunk)], idx_v)
            pltpu.async_copy(table_hbm.at[idx_v], rows_v, sem).wait()
            for j in range(GROUP):
                pltpu.sync_copy(rows_v.at[pl.ds(j * SEQ, SEQ)],
                                out_hbm.at[s0 + j])
            return carry

        lax.fori_loop(0, n_chunks, body, 0)

    return gather


def kernel(time, pe):
    n_sent = time.shape[0]
    idx = time.reshape(-1)
    assert n_sent % (NUM_WORKERS * GROUP) == 0
    n_chunks = n_sent // (NUM_WORKERS * GROUP)
    return _make_gather(n_sent, n_chunks)(pe, idx)
